# baseline (speedup 1.0000x reference)
import functools

import jax
import jax.numpy as jnp
from jax import lax
from jax.experimental import pallas as pl
from jax.experimental.pallas import tpu as pltpu
from jax.experimental.pallas import tpu_sc as plsc

WINDOW = 50
BATCH = 4096
DIM = 128
ROWS = WINDOW * BATCH
NC, NS = 2, 16
NW = NC * NS
RPW = ROWS // NW
CR = 256
NCH = RPW // CR

_MESH = plsc.VectorSubcoreMesh(core_axis_name="c", subcore_axis_name="s")


@functools.partial(
    pl.kernel,
    out_type=jax.ShapeDtypeStruct((ROWS, DIM), jnp.float32),
    mesh=_MESH,
    scratch_types=[
        pltpu.VMEM((CR, DIM), jnp.float32),
        pltpu.SemaphoreType.DMA,
        pltpu.VMEM((16,), jnp.int32),
    ],
)
def _sc_body(ptr_hbm, x_hbm, data_hbm, out_hbm, buf, sem, ptr_v):
    pltpu.sync_copy(ptr_hbm, ptr_v)
    p = ptr_v[...][0]
    wid = lax.axis_index("c") * NS + lax.axis_index("s")
    base = wid * RPW
    for c in range(NCH):
        g = base + c * CR
        slot = g // BATCH

        @pl.when(slot == p)
        def _from_x():
            pltpu.async_copy(x_hbm.at[pl.ds(g - p * BATCH, CR)], buf, sem).wait()

        @pl.when(slot != p)
        def _from_data():
            pltpu.async_copy(data_hbm.at[pl.ds(g, CR)], buf, sem).wait()

        pltpu.async_copy(buf, out_hbm.at[pl.ds(g, CR)], sem).wait()


def kernel(x, data, pointer):
    ptr = jnp.full((16,), pointer, dtype=jnp.int32)
    flat = data.reshape(ROWS, DIM)
    out = _sc_body(ptr, x, flat)
    return out.reshape(WINDOW, BATCH, DIM)


# speedup vs baseline: 1.1615x; 1.1615x over previous
import functools

import jax
import jax.numpy as jnp
from jax import lax
from jax.experimental import pallas as pl
from jax.experimental.pallas import tpu as pltpu
from jax.experimental.pallas import tpu_sc as plsc

WINDOW = 50
BATCH = 4096
DIM = 128
ROWS = WINDOW * BATCH
NC, NS = 2, 16
NW = NC * NS
RPW = ROWS // NW
CR = 256
NCH = RPW // CR

_MESH = plsc.VectorSubcoreMesh(core_axis_name="c", subcore_axis_name="s")


@functools.partial(
    pl.kernel,
    out_type=jax.ShapeDtypeStruct((ROWS, DIM), jnp.float32),
    mesh=_MESH,
    scratch_types=[
        pltpu.VMEM((CR, DIM), jnp.float32),
        pltpu.VMEM((CR, DIM), jnp.float32),
        pltpu.SemaphoreType.DMA,
        pltpu.SemaphoreType.DMA,
        pltpu.SemaphoreType.DMA,
        pltpu.SemaphoreType.DMA,
        pltpu.VMEM((16,), jnp.int32),
    ],
)
def _sc_body(ptr_hbm, x_hbm, data_hbm, out_hbm, buf0, buf1, r0, r1, w0, w1,
             ptr_v):
    pltpu.sync_copy(ptr_hbm, ptr_v)
    p = ptr_v[...][0]
    wid = lax.axis_index("c") * NS + lax.axis_index("s")
    base = wid * RPW
    bufs, rsems, wsems = (buf0, buf1), (r0, r1), (w0, w1)

    def start_read(c, buf, sem):
        g = base + c * CR
        slot = g // BATCH

        @pl.when(slot == p)
        def _from_x():
            pltpu.make_async_copy(
                x_hbm.at[pl.ds(g - p * BATCH, CR)], buf, sem).start()

        @pl.when(slot != p)
        def _from_data():
            pltpu.make_async_copy(
                data_hbm.at[pl.ds(g, CR)], buf, sem).start()

    def wait_dma(buf, sem):
        pltpu.make_async_copy(data_hbm.at[pl.ds(0, CR)], buf, sem).wait()

    start_read(0, bufs[0], rsems[0])
    for c in range(NCH):
        b = c % 2
        wait_dma(bufs[b], rsems[b])
        g = base + c * CR
        pltpu.make_async_copy(bufs[b], out_hbm.at[pl.ds(g, CR)],
                              wsems[b]).start()
        if c + 1 < NCH:
            b1 = (c + 1) % 2
            if c >= 1:
                wait_dma(bufs[b1], wsems[b1])
            start_read(c + 1, bufs[b1], rsems[b1])
    wait_dma(bufs[(NCH - 2) % 2], wsems[(NCH - 2) % 2])
    wait_dma(bufs[(NCH - 1) % 2], wsems[(NCH - 1) % 2])


def kernel(x, data, pointer):
    ptr = jnp.full((16,), pointer, dtype=jnp.int32)
    flat = data.reshape(ROWS, DIM)
    out = _sc_body(ptr, x, flat)
    return out.reshape(WINDOW, BATCH, DIM)


# speedup vs baseline: 1.1618x; 1.0003x over previous
import functools

import jax
import jax.numpy as jnp
from jax import lax
from jax.experimental import pallas as pl
from jax.experimental.pallas import tpu as pltpu
from jax.experimental.pallas import tpu_sc as plsc

WINDOW = 50
BATCH = 4096
DIM = 128
ROWS = WINDOW * BATCH
NC, NS = 2, 16
NW = NC * NS
RPW = ROWS // NW
CR = 128
NCH = RPW // CR
NBUF = 6

_MESH = plsc.VectorSubcoreMesh(core_axis_name="c", subcore_axis_name="s")

_SCRATCH = (
    [pltpu.VMEM((CR, DIM), jnp.float32) for _ in range(NBUF)]
    + [pltpu.SemaphoreType.DMA for _ in range(2 * NBUF)]
    + [pltpu.VMEM((16,), jnp.int32)]
)


@functools.partial(
    pl.kernel,
    out_type=jax.ShapeDtypeStruct((ROWS, DIM), jnp.float32),
    mesh=_MESH,
    scratch_types=_SCRATCH,
)
def _sc_body(ptr_hbm, x_hbm, data_hbm, out_hbm, *scratch):
    bufs = scratch[:NBUF]
    rsems = scratch[NBUF:2 * NBUF]
    wsems = scratch[2 * NBUF:3 * NBUF]
    ptr_v = scratch[3 * NBUF]

    pltpu.sync_copy(ptr_hbm, ptr_v)
    p = ptr_v[...][0]
    wid = lax.axis_index("c") * NS + lax.axis_index("s")
    base = wid * RPW

    def start_read(c, buf, sem):
        g = base + c * CR
        slot = g // BATCH

        @pl.when(slot == p)
        def _from_x():
            pltpu.make_async_copy(
                x_hbm.at[pl.ds(g - p * BATCH, CR)], buf, sem).start()

        @pl.when(slot != p)
        def _from_data():
            pltpu.make_async_copy(
                data_hbm.at[pl.ds(g, CR)], buf, sem).start()

    def wait_dma(buf, sem):
        pltpu.make_async_copy(data_hbm.at[pl.ds(0, CR)], buf, sem).wait()

    for c in range(min(NBUF - 1, NCH)):
        start_read(c, bufs[c % NBUF], rsems[c % NBUF])
    for c in range(NCH):
        b = c % NBUF
        wait_dma(bufs[b], rsems[b])
        g = base + c * CR
        pltpu.make_async_copy(bufs[b], out_hbm.at[pl.ds(g, CR)],
                              wsems[b]).start()
        nxt = c + NBUF - 1
        if nxt < NCH:
            bn = nxt % NBUF
            if nxt >= NBUF:
                wait_dma(bufs[bn], wsems[bn])
            start_read(nxt, bufs[bn], rsems[bn])
    for c in range(max(0, NCH - NBUF), NCH):
        b = c % NBUF
        wait_dma(bufs[b], wsems[b])


def kernel(x, data, pointer):
    ptr = jnp.full((16,), pointer, dtype=jnp.int32)
    flat = data.reshape(ROWS, DIM)
    out = _sc_body(ptr, x, flat)
    return out.reshape(WINDOW, BATCH, DIM)


# speedup vs baseline: 1.7018x; 1.4647x over previous
import jax
import jax.numpy as jnp
from jax.experimental import pallas as pl
from jax.experimental.pallas import tpu as pltpu

WINDOW = 50
BATCH = 4096
DIM = 128
SLOTS = 5
NCHUNK = WINDOW // SLOTS
NBUF = 4


def _body(ptr_ref, x_ref, data_ref, out_ref, *scratch):
    bufs = scratch[:NBUF]
    rsems = scratch[NBUF:2 * NBUF]
    wsems = scratch[2 * NBUF:3 * NBUF]
    p = ptr_ref[0]

    def start_reads(c, buf, sem):
        for s in range(SLOTS):
            slot = c * SLOTS + s

            @pl.when(slot == p)
            def _from_x():
                pltpu.make_async_copy(
                    x_ref, buf.at[pl.ds(s * BATCH, BATCH)], sem).start()

            @pl.when(slot != p)
            def _from_data():
                pltpu.make_async_copy(
                    data_ref.at[pl.ds(slot * BATCH, BATCH)],
                    buf.at[pl.ds(s * BATCH, BATCH)], sem).start()

    def wait_reads(buf, sem):
        for s in range(SLOTS):
            pltpu.make_async_copy(
                data_ref.at[pl.ds(0, BATCH)],
                buf.at[pl.ds(s * BATCH, BATCH)], sem).wait()

    def wait_write(buf, sem):
        pltpu.make_async_copy(buf, out_ref.at[pl.ds(0, SLOTS * BATCH)],
                              sem).wait()

    for c in range(NBUF - 1):
        start_reads(c, bufs[c % NBUF], rsems[c % NBUF])
    for c in range(NCHUNK):
        b = c % NBUF
        wait_reads(bufs[b], rsems[b])
        pltpu.make_async_copy(
            bufs[b], out_ref.at[pl.ds(c * SLOTS * BATCH, SLOTS * BATCH)],
            wsems[b]).start()
        nxt = c + NBUF - 1
        if nxt < NCHUNK:
            bn = nxt % NBUF
            if nxt >= NBUF:
                wait_write(bufs[bn], wsems[bn])
            start_reads(nxt, bufs[bn], rsems[bn])
    for c in range(max(0, NCHUNK - NBUF), NCHUNK):
        b = c % NBUF
        wait_write(bufs[b], wsems[b])


def kernel(x, data, pointer):
    ptr = jnp.atleast_1d(jnp.asarray(pointer, dtype=jnp.int32))
    flat = data.reshape(WINDOW * BATCH, DIM)
    grid_spec = pltpu.PrefetchScalarGridSpec(
        num_scalar_prefetch=1,
        grid=(1,),
        in_specs=[
            pl.BlockSpec(memory_space=pl.MemorySpace.ANY),
            pl.BlockSpec(memory_space=pl.MemorySpace.ANY),
        ],
        out_specs=pl.BlockSpec(memory_space=pl.MemorySpace.ANY),
        scratch_shapes=(
            [pltpu.VMEM((SLOTS * BATCH, DIM), jnp.float32)
             for _ in range(NBUF)]
            + [pltpu.SemaphoreType.DMA for _ in range(2 * NBUF)]
        ),
    )
    out = pl.pallas_call(
        _body,
        grid_spec=grid_spec,
        out_shape=jax.ShapeDtypeStruct((WINDOW * BATCH, DIM), jnp.float32),
    )(ptr, x, flat)
    return out.reshape(WINDOW, BATCH, DIM)


# speedup vs baseline: 1.7086x; 1.0040x over previous
import jax
import jax.numpy as jnp
from jax.experimental import pallas as pl
from jax.experimental.pallas import tpu as pltpu

WINDOW = 50
BATCH = 4096
DIM = 128
SLOTS = 2
NCHUNK = WINDOW // SLOTS
NBUF = 10


def _body(ptr_ref, x_ref, data_ref, out_ref, *scratch):
    bufs = scratch[:NBUF]
    rsems = scratch[NBUF:2 * NBUF]
    wsems = scratch[2 * NBUF:3 * NBUF]
    p = ptr_ref[0]

    def start_reads(c, buf, sem):
        for s in range(SLOTS):
            slot = c * SLOTS + s

            @pl.when(slot == p)
            def _from_x():
                pltpu.make_async_copy(
                    x_ref, buf.at[pl.ds(s * BATCH, BATCH)], sem).start()

            @pl.when(slot != p)
            def _from_data():
                pltpu.make_async_copy(
                    data_ref.at[pl.ds(slot * BATCH, BATCH)],
                    buf.at[pl.ds(s * BATCH, BATCH)], sem).start()

    def wait_reads(buf, sem):
        for s in range(SLOTS):
            pltpu.make_async_copy(
                data_ref.at[pl.ds(0, BATCH)],
                buf.at[pl.ds(s * BATCH, BATCH)], sem).wait()

    def wait_write(buf, sem):
        pltpu.make_async_copy(buf, out_ref.at[pl.ds(0, SLOTS * BATCH)],
                              sem).wait()

    for c in range(NBUF - 1):
        start_reads(c, bufs[c % NBUF], rsems[c % NBUF])
    for c in range(NCHUNK):
        b = c % NBUF
        wait_reads(bufs[b], rsems[b])
        pltpu.make_async_copy(
            bufs[b], out_ref.at[pl.ds(c * SLOTS * BATCH, SLOTS * BATCH)],
            wsems[b]).start()
        nxt = c + NBUF - 1
        if nxt < NCHUNK:
            bn = nxt % NBUF
            if nxt >= NBUF:
                wait_write(bufs[bn], wsems[bn])
            start_reads(nxt, bufs[bn], rsems[bn])
    for c in range(max(0, NCHUNK - NBUF), NCHUNK):
        b = c % NBUF
        wait_write(bufs[b], wsems[b])


def kernel(x, data, pointer):
    ptr = jnp.atleast_1d(jnp.asarray(pointer, dtype=jnp.int32))
    flat = data.reshape(WINDOW * BATCH, DIM)
    grid_spec = pltpu.PrefetchScalarGridSpec(
        num_scalar_prefetch=1,
        grid=(1,),
        in_specs=[
            pl.BlockSpec(memory_space=pl.MemorySpace.ANY),
            pl.BlockSpec(memory_space=pl.MemorySpace.ANY),
        ],
        out_specs=pl.BlockSpec(memory_space=pl.MemorySpace.ANY),
        scratch_shapes=(
            [pltpu.VMEM((SLOTS * BATCH, DIM), jnp.float32)
             for _ in range(NBUF)]
            + [pltpu.SemaphoreType.DMA for _ in range(2 * NBUF)]
        ),
    )
    out = pl.pallas_call(
        _body,
        grid_spec=grid_spec,
        out_shape=jax.ShapeDtypeStruct((WINDOW * BATCH, DIM), jnp.float32),
    )(ptr, x, flat)
    return out.reshape(WINDOW, BATCH, DIM)
